# baseline (speedup 1.0000x reference)
import functools

import jax
import jax.numpy as jnp
from jax import lax
from jax.experimental import pallas as pl
from jax.experimental.pallas import tpu as pltpu
from jax.experimental.pallas import tpu_sc as plsc

VOCAB = 100000
MAXLEN = 2048
EMBED = 64
BATCH = 64
SEQ = 2048

NUM_CORES = 2
NUM_SUBCORES = 16
NW = NUM_CORES * NUM_SUBCORES
EPW = EMBED // NW
LANES = 16
NSTEP = SEQ // LANES
NBUF = 3


def _make_kernel():
    mesh = plsc.VectorSubcoreMesh(core_axis_name="c", subcore_axis_name="s")

    @functools.partial(
        pl.kernel,
        mesh=mesh,
        out_type=jax.ShapeDtypeStruct((BATCH, EMBED, SEQ), jnp.float32),
        compiler_params=pltpu.CompilerParams(
            use_tc_tiling_on_sc=True, needs_layout_passes=False),
        scratch_types=[
            pltpu.VMEM((VOCAB,), jnp.float32),
            pltpu.VMEM((SEQ,), jnp.int32),
            pltpu.VMEM((SEQ,), jnp.int32),
            pltpu.VMEM((SEQ,), jnp.int32),
            pltpu.VMEM((SEQ,), jnp.float32),
            pltpu.VMEM((SEQ,), jnp.float32),
            pltpu.VMEM((SEQ,), jnp.float32),
        ]
        + [pltpu.SemaphoreType.DMA] * (3 * NBUF + 1),
    )
    def emb(x_hbm, tokT_hbm, posT_hbm, outT_hbm, row_v, xv0, xv1, xv2,
            ov0, ov1, ov2, *sems):
        xbufs = (xv0, xv1, xv2)
        obufs = (ov0, ov1, ov2)
        xsem = sems[0:NBUF]
        osem = sems[NBUF : 2 * NBUF]
        psem = sems[2 * NBUF : 3 * NBUF]
        rsem = sems[3 * NBUF]
        c = lax.axis_index("c")
        s = lax.axis_index("s")
        wid = s * NUM_CORES + c

        def per_component(t, carry):
            e = wid * EPW + t
            row_cp = pltpu.async_copy(tokT_hbm.at[e], row_v, rsem)

            def fetch_x(b):
                return pltpu.async_copy(
                    x_hbm.at[pl.ds(b * SEQ, SEQ)], xbufs[b % NBUF],
                    xsem[b % NBUF])

            def fetch_pos(b):
                return pltpu.async_copy(posT_hbm.at[e], obufs[b % NBUF],
                                        psem[b % NBUF])

            xfetch = {0: fetch_x(0), 1: fetch_x(1)}
            pfill = {0: fetch_pos(0), 1: fetch_pos(1)}
            row_cp.wait()

            stores = {}
            for b in range(BATCH):
                xb = b % NBUF
                xfetch[b].wait()
                pfill[b].wait()
                if b + 2 < BATCH:
                    xfetch[b + 2] = fetch_x(b + 2)
                if b - (NBUF - 1) >= 0:
                    stores[b - (NBUF - 1)].wait()
                if b + 1 < BATCH:
                    pfill[b + 1] = fetch_pos(b + 1)
                xr = xbufs[xb]
                orow = obufs[xb]
                pltpu.sync_copy(posT_hbm.at[e], orow)

                @plsc.parallel_loop(0, NSTEP, unroll=8)
                def sbody(i):
                    sl = pl.ds(i * LANES, LANES)
                    g = plsc.load_gather(row_v, [xr[sl]])
                    orow[sl] = orow[sl] + g

                stores[b] = pltpu.async_copy(orow, outT_hbm.at[b, e, :],
                                             osem[xb])
            for b in range(BATCH - (NBUF - 1), BATCH):
                stores[b].wait()
            return carry

        lax.fori_loop(0, EPW, per_component, 0)

    return emb


_emb = _make_kernel()


def kernel(x, token_table, pos_table):
    x_flat = x.reshape(BATCH * SEQ).astype(jnp.int32)
    outT = _emb(x_flat, token_table.T, pos_table.T)
    return outT.transpose(0, 2, 1)


# speedup vs baseline: 2.1011x; 2.1011x over previous
import functools

import jax
import jax.numpy as jnp
from jax import lax
from jax.experimental import pallas as pl
from jax.experimental.pallas import tpu as pltpu
from jax.experimental.pallas import tpu_sc as plsc

VOCAB = 100000
MAXLEN = 2048
EMBED = 64
BATCH = 64
SEQ = 2048

NUM_CORES = 2
NUM_SUBCORES = 16
NW = NUM_CORES * NUM_SUBCORES
EPW = EMBED // NW
LANES = 16
NSTEP = SEQ // LANES
PAIRS = BATCH // 2


def _make_kernel():
    mesh = plsc.VectorSubcoreMesh(core_axis_name="c", subcore_axis_name="s")

    @functools.partial(
        pl.kernel,
        mesh=mesh,
        out_type=jax.ShapeDtypeStruct((BATCH, EMBED, SEQ), jnp.float32),
        compiler_params=pltpu.CompilerParams(
            use_tc_tiling_on_sc=True, needs_layout_passes=False),
        scratch_types=[
            pltpu.VMEM((VOCAB,), jnp.float32),
            pltpu.VMEM((SEQ,), jnp.int32),
            pltpu.VMEM((SEQ,), jnp.int32),
            pltpu.VMEM((SEQ,), jnp.int32),
            pltpu.VMEM((SEQ,), jnp.int32),
            pltpu.VMEM((SEQ,), jnp.float32),
            pltpu.VMEM((SEQ,), jnp.float32),
            pltpu.VMEM((SEQ,), jnp.float32),
            pltpu.VMEM((SEQ,), jnp.float32),
            pltpu.VMEM((SEQ,), jnp.float32),
        ]
        + [pltpu.SemaphoreType.DMA] * 10,
    )
    def emb(x_hbm, tokT_hbm, posT_hbm, outT_hbm, row_v, xv0, xv1, xv2, xv3,
            ov0, ov1, ov2, ov3, pos_r, *sems):
        xbufs = (xv0, xv1, xv2, xv3)
        obufs = (ov0, ov1, ov2, ov3)
        xsem = sems[0:4]
        osem = sems[4:8]
        rsem = sems[8]
        psem = sems[9]
        c = lax.axis_index("c")
        s = lax.axis_index("s")
        wid = s * NUM_CORES + c

        def per_component(t, carry):
            e = wid * EPW + t
            row_cp = pltpu.async_copy(tokT_hbm.at[e], row_v, rsem)
            pos_cp = pltpu.async_copy(posT_hbm.at[e], pos_r, psem)

            def fetch_x(b, slot):
                return pltpu.async_copy(
                    x_hbm.at[pl.ds(b * SEQ, SEQ)], xbufs[slot], xsem[slot])

            def fetch_pair(p):
                s0 = (p % 2) * 2
                return (fetch_x(2 * p, s0), fetch_x(2 * p + 1, s0 + 1))

            xfetch = {0: fetch_pair(0), 1: fetch_pair(1)}
            row_cp.wait()
            pos_cp.wait()

            stores = {}
            for p in range(PAIRS):
                s0 = (p % 2) * 2
                xfetch[p][0].wait()
                xfetch[p][1].wait()
                if p - 2 >= 0:
                    stores[p - 2][0].wait()
                    stores[p - 2][1].wait()
                x0r, x1r = xbufs[s0], xbufs[s0 + 1]
                o0r, o1r = obufs[s0], obufs[s0 + 1]

                @plsc.parallel_loop(0, NSTEP, unroll=4)
                def sbody(i):
                    sl = pl.ds(i * LANES, LANES)
                    pv = pos_r[sl]
                    g0 = plsc.load_gather(row_v, [x0r[sl]])
                    g1 = plsc.load_gather(row_v, [x1r[sl]])
                    o0r[sl] = g0 + pv
                    o1r[sl] = g1 + pv

                if p + 2 < PAIRS:
                    xfetch[p + 2] = fetch_pair(p + 2)
                stores[p] = (
                    pltpu.async_copy(o0r, outT_hbm.at[2 * p, e, :], osem[s0]),
                    pltpu.async_copy(o1r, outT_hbm.at[2 * p + 1, e, :],
                                     osem[s0 + 1]),
                )
            for p in range(PAIRS - 2, PAIRS):
                stores[p][0].wait()
                stores[p][1].wait()
            return carry

        lax.fori_loop(0, EPW, per_component, 0)

    return emb


_emb = _make_kernel()


def kernel(x, token_table, pos_table):
    x_flat = x.reshape(BATCH * SEQ).astype(jnp.int32)
    outT = _emb(x_flat, token_table.T, pos_table.T)
    return outT.transpose(0, 2, 1)
